# baseline (speedup 1.0000x reference)
import functools

import jax
import jax.numpy as jnp
from jax import lax
from jax.experimental import pallas as pl
from jax.experimental.pallas import tpu as pltpu
from jax.experimental.pallas import tpu_sc as plsc

_NUM_EMB = 1024
_EMB_DIM = 64
_B, _C, _H, _W = 16, 64, 32, 32
_N = _B * _H * _W
_ROWS = 4096
_STEPS = _N // _ROWS
_COMMIT = 0.25

_NC, _NS = 2, 16
_NW = _NC * _NS
_BPW = _N // _NW


def _vq_body(x_ref, w_ref, enc_ref, idx_ref, loss_ref, perp_ref,
             counts_ref, acc_ref):
    i = pl.program_id(0)
    x = x_ref[...]
    w = w_ref[...]

    a = jnp.sum(x ** 2, axis=1, keepdims=True)
    b = jnp.sum(w ** 2, axis=1)
    c2 = lax.dot_general(x, w + w, (((1,), (1,)), ((), ())),
                         preferred_element_type=jnp.float32)
    d = a + b - c2

    dmin = jnp.min(d, axis=1, keepdims=True)
    iota = lax.broadcasted_iota(jnp.int32, (_ROWS, _NUM_EMB), 1
                                ).astype(jnp.float32)
    idx = jnp.min(jnp.where(d == dmin, iota, jnp.float32(_NUM_EMB)),
                  axis=1)
    onehot = (iota == idx[:, None]).astype(jnp.float32)
    enc_ref[...] = onehot
    idx_ref[...] = idx.astype(jnp.int32).reshape(1, _ROWS // _NUM_EMB,
                                                  _NUM_EMB)

    @pl.when(i == 0)
    def _init():
        acc_ref[0, 0] = 0.0
        counts_ref[...] = jnp.zeros_like(counts_ref)

    acc_ref[0, 0] += jnp.sum(dmin)
    ones = jnp.ones((1, _ROWS), jnp.float32)
    counts_ref[...] += lax.dot_general(ones, onehot, (((1,), (0,)), ((), ())),
                                       preferred_element_type=jnp.float32)

    @pl.when(i == pl.num_programs(0) - 1)
    def _fin():
        m = acc_ref[0, 0] / (_N * _EMB_DIM)
        loss_ref[...] = jnp.reshape(m + _COMMIT * m, (1, 1))
        avg = counts_ref[...] / _N
        ent = jnp.sum(avg * jnp.log(avg + 1e-10))
        perp_ref[...] = jnp.reshape(jnp.exp(-ent), (1, 1))


_SC_MESH = plsc.VectorSubcoreMesh(core_axis_name="c", subcore_axis_name="s")


@functools.partial(
    pl.kernel, mesh=_SC_MESH,
    out_type=jax.ShapeDtypeStruct((_N, _EMB_DIM), jnp.float32),
    scratch_types=[
        pltpu.VMEM((_BPW,), jnp.int32),
        pltpu.VMEM((_BPW, _EMB_DIM), jnp.float32),
        pltpu.SemaphoreType.DMA,
    ],
    compiler_params=pltpu.CompilerParams(use_tc_tiling_on_sc=False),
)
def _sc_gather(w_hbm, idx_hbm, out_hbm, idx_v, rows_v, sem):
    wid = lax.axis_index("s") * _NC + lax.axis_index("c")
    base = wid * _BPW
    pltpu.sync_copy(idx_hbm.at[pl.ds(base, _BPW)], idx_v)
    pltpu.async_copy(w_hbm.at[idx_v], rows_v, sem).wait()
    pltpu.sync_copy(rows_v, out_hbm.at[pl.ds(base, _BPW)])


def kernel(inp, W):
    x = jnp.transpose(inp, (0, 2, 3, 1))
    flat = x.reshape(_N, _EMB_DIM)
    enc, idx2d, loss, perp = pl.pallas_call(
        _vq_body,
        grid=(_STEPS,),
        in_specs=[
            pl.BlockSpec((_ROWS, _EMB_DIM), lambda i: (i, 0)),
            pl.BlockSpec((_NUM_EMB, _EMB_DIM), lambda i: (0, 0)),
        ],
        out_specs=[
            pl.BlockSpec((_ROWS, _NUM_EMB), lambda i: (i, 0)),
            pl.BlockSpec((1, _ROWS // _NUM_EMB, _NUM_EMB),
                         lambda i: (i, 0, 0)),
            pl.BlockSpec((1, 1), lambda i: (0, 0)),
            pl.BlockSpec((1, 1), lambda i: (0, 0)),
        ],
        out_shape=[
            jax.ShapeDtypeStruct((_N, _NUM_EMB), jnp.float32),
            jax.ShapeDtypeStruct((_STEPS, _ROWS // _NUM_EMB, _NUM_EMB),
                                 jnp.int32),
            jax.ShapeDtypeStruct((1, 1), jnp.float32),
            jax.ShapeDtypeStruct((1, 1), jnp.float32),
        ],
        scratch_shapes=[
            pltpu.VMEM((1, _NUM_EMB), jnp.float32),
            pltpu.SMEM((1, 1), jnp.float32),
        ],
        compiler_params=pltpu.CompilerParams(
            dimension_semantics=("arbitrary",)),
    )(flat, W)
    e_flat = _sc_gather(W, idx2d.reshape(_N))
    e_out = jnp.transpose(e_flat.reshape(_B, _H, _W, _C), (0, 3, 1, 2))
    return (loss[0, 0], e_out, perp[0, 0], enc)


# speedup vs baseline: 1.4781x; 1.4781x over previous
import jax
import jax.numpy as jnp
from jax import lax
from jax.experimental import pallas as pl
from jax.experimental.pallas import tpu as pltpu

_NUM_EMB = 1024
_EMB_DIM = 64
_B, _C, _H, _W = 16, 64, 32, 32
_N = _B * _H * _W
_ROWS = 4096
_STEPS = _N // _ROWS
_COMMIT = 0.25


def _vq_body(x_ref, w_ref, enc_ref, e_ref, loss_ref, perp_ref,
             counts_ref, acc_ref):
    i = pl.program_id(0)
    x = x_ref[...]
    w = w_ref[...]

    a = jnp.sum(x ** 2, axis=1, keepdims=True)
    b = jnp.sum(w ** 2, axis=1)
    c2 = lax.dot_general(x, w + w, (((1,), (1,)), ((), ())),
                         preferred_element_type=jnp.float32)
    d = a + b - c2

    dmin = jnp.min(d, axis=1, keepdims=True)
    iota = lax.broadcasted_iota(jnp.int32, (_ROWS, _NUM_EMB), 1
                                ).astype(jnp.float32)
    idx = jnp.min(jnp.where(d == dmin, iota, jnp.float32(_NUM_EMB)),
                  axis=1)
    onehot = (iota == idx[:, None]).astype(jnp.float32)
    enc_ref[...] = onehot

    e_ref[...] = lax.dot_general(onehot, w, (((1,), (0,)), ((), ())),
                                 preferred_element_type=jnp.float32)

    @pl.when(i == 0)
    def _init():
        acc_ref[0, 0] = 0.0
        counts_ref[...] = jnp.zeros_like(counts_ref)

    acc_ref[0, 0] += jnp.sum(dmin)
    ones = jnp.ones((1, _ROWS), jnp.float32)
    counts_ref[...] += lax.dot_general(ones, onehot, (((1,), (0,)), ((), ())),
                                       preferred_element_type=jnp.float32)

    @pl.when(i == pl.num_programs(0) - 1)
    def _fin():
        m = acc_ref[0, 0] / (_N * _EMB_DIM)
        loss_ref[...] = jnp.reshape(m + _COMMIT * m, (1, 1))
        avg = counts_ref[...] / _N
        ent = jnp.sum(avg * jnp.log(avg + 1e-10))
        perp_ref[...] = jnp.reshape(jnp.exp(-ent), (1, 1))


def kernel(inp, W):
    x = jnp.transpose(inp, (0, 2, 3, 1))
    flat = x.reshape(_N, _EMB_DIM)
    enc, e_flat, loss, perp = pl.pallas_call(
        _vq_body,
        grid=(_STEPS,),
        in_specs=[
            pl.BlockSpec((_ROWS, _EMB_DIM), lambda i: (i, 0)),
            pl.BlockSpec((_NUM_EMB, _EMB_DIM), lambda i: (0, 0)),
        ],
        out_specs=[
            pl.BlockSpec((_ROWS, _NUM_EMB), lambda i: (i, 0)),
            pl.BlockSpec((_ROWS, _EMB_DIM), lambda i: (i, 0)),
            pl.BlockSpec((1, 1), lambda i: (0, 0)),
            pl.BlockSpec((1, 1), lambda i: (0, 0)),
        ],
        out_shape=[
            jax.ShapeDtypeStruct((_N, _NUM_EMB), jnp.float32),
            jax.ShapeDtypeStruct((_N, _EMB_DIM), jnp.float32),
            jax.ShapeDtypeStruct((1, 1), jnp.float32),
            jax.ShapeDtypeStruct((1, 1), jnp.float32),
        ],
        scratch_shapes=[
            pltpu.VMEM((1, _NUM_EMB), jnp.float32),
            pltpu.SMEM((1, 1), jnp.float32),
        ],
        compiler_params=pltpu.CompilerParams(
            dimension_semantics=("arbitrary",)),
    )(flat, W)
    e_out = jnp.transpose(e_flat.reshape(_B, _H, _W, _C), (0, 3, 1, 2))
    return (loss[0, 0], e_out, perp[0, 0], enc)


# speedup vs baseline: 1.4792x; 1.0007x over previous
import jax
import jax.numpy as jnp
from jax import lax
from jax.experimental import pallas as pl
from jax.experimental.pallas import tpu as pltpu

_NUM_EMB = 1024
_EMB_DIM = 64
_B, _C, _H, _W = 16, 64, 32, 32
_N = _B * _H * _W
_ROWS = 2048
_STEPS = _N // _ROWS
_COMMIT = 0.25


def _vq_body(x_ref, w_ref, enc_ref, e_ref, loss_ref, perp_ref,
             counts_ref, acc_ref):
    i = pl.program_id(0)
    x = x_ref[...]
    w = w_ref[...]

    a = jnp.sum(x ** 2, axis=1, keepdims=True)
    b = jnp.sum(w ** 2, axis=1)
    c2 = lax.dot_general(x, w + w, (((1,), (1,)), ((), ())),
                         preferred_element_type=jnp.float32)
    d = a + b - c2

    dmin = jnp.min(d, axis=1, keepdims=True)
    iota = lax.broadcasted_iota(jnp.int32, (_ROWS, _NUM_EMB), 1
                                ).astype(jnp.float32)
    idx = jnp.min(jnp.where(d == dmin, iota, jnp.float32(_NUM_EMB)),
                  axis=1)
    onehot = (iota == idx[:, None]).astype(jnp.float32)
    enc_ref[...] = onehot

    e_ref[...] = lax.dot_general(onehot, w, (((1,), (0,)), ((), ())),
                                 preferred_element_type=jnp.float32)

    @pl.when(i == 0)
    def _init():
        acc_ref[0, 0] = 0.0
        counts_ref[...] = jnp.zeros_like(counts_ref)

    acc_ref[0, 0] += jnp.sum(dmin)
    ones = jnp.ones((1, _ROWS), jnp.float32)
    counts_ref[...] += lax.dot_general(ones, onehot, (((1,), (0,)), ((), ())),
                                       preferred_element_type=jnp.float32)

    @pl.when(i == pl.num_programs(0) - 1)
    def _fin():
        m = acc_ref[0, 0] / (_N * _EMB_DIM)
        loss_ref[...] = jnp.reshape(m + _COMMIT * m, (1, 1))
        avg = counts_ref[...] / _N
        ent = jnp.sum(avg * jnp.log(avg + 1e-10))
        perp_ref[...] = jnp.reshape(jnp.exp(-ent), (1, 1))


def kernel(inp, W):
    x = jnp.transpose(inp, (0, 2, 3, 1))
    flat = x.reshape(_N, _EMB_DIM)
    enc, e_flat, loss, perp = pl.pallas_call(
        _vq_body,
        grid=(_STEPS,),
        in_specs=[
            pl.BlockSpec((_ROWS, _EMB_DIM), lambda i: (i, 0)),
            pl.BlockSpec((_NUM_EMB, _EMB_DIM), lambda i: (0, 0)),
        ],
        out_specs=[
            pl.BlockSpec((_ROWS, _NUM_EMB), lambda i: (i, 0)),
            pl.BlockSpec((_ROWS, _EMB_DIM), lambda i: (i, 0)),
            pl.BlockSpec((1, 1), lambda i: (0, 0)),
            pl.BlockSpec((1, 1), lambda i: (0, 0)),
        ],
        out_shape=[
            jax.ShapeDtypeStruct((_N, _NUM_EMB), jnp.float32),
            jax.ShapeDtypeStruct((_N, _EMB_DIM), jnp.float32),
            jax.ShapeDtypeStruct((1, 1), jnp.float32),
            jax.ShapeDtypeStruct((1, 1), jnp.float32),
        ],
        scratch_shapes=[
            pltpu.VMEM((1, _NUM_EMB), jnp.float32),
            pltpu.SMEM((1, 1), jnp.float32),
        ],
        compiler_params=pltpu.CompilerParams(
            dimension_semantics=("arbitrary",)),
    )(flat, W)
    e_out = jnp.transpose(e_flat.reshape(_B, _H, _W, _C), (0, 3, 1, 2))
    return (loss[0, 0], e_out, perp[0, 0], enc)
